# baseline (speedup 1.0000x reference)
import functools

import jax
import jax.numpy as jnp
from jax import lax
from jax.experimental import pallas as pl
from jax.experimental.pallas import tpu as pltpu
from jax.experimental.pallas import tpu_sc as plsc

B, S, D = 8, 2048, 8
VOCAB = 100000
D_MODEL = 256
BS = B * S

NC, NS, L = 2, 16, 16
NW = NC * NS
TW = BS // NW
C = 16
SLOTS = C * D
NCH = TW // C
NV = D_MODEL // L
SL2, LANES = 2, 128
ZROW = SLOTS + D_MODEL


def _sc_pooled(ids, tys, lens, key_table, index_table):
    mesh = plsc.VectorSubcoreMesh(core_axis_name="c", subcore_axis_name="s")

    @functools.partial(
        pl.kernel,
        out_type=jax.ShapeDtypeStruct((BS * D_MODEL,), jnp.float32),
        mesh=mesh,
        scratch_types=[
            pltpu.VMEM((SLOTS,), jnp.int32),
            pltpu.VMEM((SLOTS,), jnp.int32),
            pltpu.VMEM((SLOTS,), jnp.int32),
            pltpu.VMEM((SLOTS,), jnp.int32),
            pltpu.VMEM((SLOTS,), jnp.int32),
            pltpu.VMEM((ZROW + 8, SL2, LANES), jnp.float32),
            pltpu.VMEM((C * D_MODEL,), jnp.float32),
        ],
    )
    def k(ids_hbm, tys_hbm, lens_hbm, ktab_hbm, itab_hbm, out_hbm,
          ids_v, tys_v, lens_v, kidx_v, sr_v, rows_v, out_v):
        wid = lax.axis_index("s") * NC + lax.axis_index("c")
        lane = lax.iota(jnp.int32, L)
        pos = lax.rem(lane, D)
        zeros = jnp.zeros((L,), jnp.float32)
        pltpu.sync_copy(itab_hbm, rows_v.at[pl.ds(SLOTS, D_MODEL)])
        for h in range(SL2):
            for hh in range(LANES // L):
                rows_v[ZROW, h, pl.ds(hh * L, L)] = zeros

        @pl.loop(0, NCH)
        def _(ch):
            tok0 = wid * TW + ch * C
            s0 = tok0 * D
            pltpu.sync_copy(ids_hbm.at[pl.ds(s0, SLOTS)], ids_v)
            pltpu.sync_copy(tys_hbm.at[pl.ds(s0, SLOTS)], tys_v)
            pltpu.sync_copy(lens_hbm.at[pl.ds(s0, SLOTS)], lens_v)

            for g in range(SLOTS // L):
                sl = pl.ds(g * L, L)
                idv = ids_v[sl]
                tyv = tys_v[sl]
                valid = pos < lens_v[sl]
                km = valid & (tyv == 1)
                im = valid & (tyv == 2)
                slot = lane + (g * L)
                kidx_v[sl] = jnp.where(km, idv, 0)
                sr_v[sl] = jnp.where(
                    km, slot,
                    jnp.where(im, SLOTS + jnp.minimum(idv, D_MODEL - 1), ZROW))

            pltpu.sync_copy(ktab_hbm.at[kidx_v], rows_v.at[pl.ds(0, SLOTS)])

            @pl.loop(0, SLOTS // L)
            def _(g):
                srv = sr_v[pl.ds(g * L, L)]
                for half in range(L // D):
                    t = g * (L // D) + half
                    r0 = srv[half * D]
                    acc = [rows_v[r0, v // (LANES // L), pl.ds((v % (LANES // L)) * L, L)]
                           for v in range(NV)]
                    for d in range(1, D):
                        r = srv[half * D + d]
                        for v in range(NV):
                            acc[v] = acc[v] + rows_v[
                                r, v // (LANES // L), pl.ds((v % (LANES // L)) * L, L)]
                    for v in range(NV):
                        out_v[pl.ds(t * D_MODEL + v * L, L)] = acc[v]

            pltpu.sync_copy(out_v, out_hbm.at[pl.ds(tok0 * D_MODEL,
                                                    C * D_MODEL)])

    return k(ids, tys, lens, key_table, index_table)


@jax.jit
def kernel(path_types, path_ids, path_lengths, key_table, index_table):
    ids = path_ids.reshape(-1).astype(jnp.int32)
    tys = path_types.reshape(-1).astype(jnp.int32)
    lens = jnp.broadcast_to(
        path_lengths.astype(jnp.int32)[..., None], (B, S, D)
    ).reshape(-1)
    out = _sc_pooled(ids, tys, lens,
                     key_table.astype(jnp.float32).reshape(VOCAB, SL2, LANES),
                     index_table.astype(jnp.float32).reshape(
                         D_MODEL, SL2, LANES))
    return out.reshape(B, S, D_MODEL)


# speedup vs baseline: 1.0267x; 1.0267x over previous
import functools

import jax
import jax.numpy as jnp
from jax import lax
from jax.experimental import pallas as pl
from jax.experimental.pallas import tpu as pltpu
from jax.experimental.pallas import tpu_sc as plsc

B, S, D = 8, 2048, 8
D_MODEL = 256
BS = B * S

NC, NS, L = 2, 16, 16
NW = NC * NS
TW = BS // NW
C = 16
SLOTS = C * D
NCH = TW // C
NV = D_MODEL // L
ZROW = SLOTS + D_MODEL


def _sc_pooled(ids, tys, lens, key_table, index_table):
    mesh = plsc.VectorSubcoreMesh(core_axis_name="c", subcore_axis_name="s")

    @functools.partial(
        pl.kernel,
        out_type=jax.ShapeDtypeStruct((BS * D_MODEL,), jnp.float32),
        mesh=mesh,
        compiler_params=pltpu.CompilerParams(use_tc_tiling_on_sc=True),
        scratch_types=[
            pltpu.VMEM((SLOTS,), jnp.int32),
            pltpu.VMEM((SLOTS,), jnp.int32),
            pltpu.VMEM((SLOTS,), jnp.int32),
            pltpu.VMEM((SLOTS,), jnp.int32),
            pltpu.VMEM((SLOTS,), jnp.int32),
            pltpu.VMEM((ZROW + 8, D_MODEL), jnp.float32),
            pltpu.VMEM((C * D_MODEL,), jnp.float32),
        ],
    )
    def k(ids_hbm, tys_hbm, lens_hbm, ktab_hbm, itab_hbm, out_hbm,
          ids_v, tys_v, lens_v, kidx_v, sr_v, rows_v, out_v):
        wid = lax.axis_index("s") * NC + lax.axis_index("c")
        lane = lax.iota(jnp.int32, L)
        pos = lax.rem(lane, D)
        zeros = jnp.zeros((L,), jnp.float32)
        pltpu.sync_copy(itab_hbm, rows_v.at[pl.ds(SLOTS, D_MODEL)])
        for v in range(NV):
            rows_v[ZROW, pl.ds(v * L, L)] = zeros

        @pl.loop(0, NCH)
        def _(ch):
            tok0 = wid * TW + ch * C
            s0 = tok0 * D
            pltpu.sync_copy(ids_hbm.at[pl.ds(s0, SLOTS)], ids_v)
            pltpu.sync_copy(tys_hbm.at[pl.ds(s0, SLOTS)], tys_v)
            pltpu.sync_copy(lens_hbm.at[pl.ds(s0, SLOTS)], lens_v)

            for g in range(SLOTS // L):
                sl = pl.ds(g * L, L)
                idv = ids_v[sl]
                tyv = tys_v[sl]
                valid = pos < lens_v[sl]
                km = valid & (tyv == 1)
                im = valid & (tyv == 2)
                slot = lane + (g * L)
                kidx_v[sl] = jnp.where(km, idv, 0)
                sr_v[sl] = jnp.where(
                    km, slot,
                    jnp.where(im, SLOTS + jnp.minimum(idv, D_MODEL - 1), ZROW))

            pltpu.sync_copy(ktab_hbm.at[kidx_v], rows_v.at[pl.ds(0, SLOTS)])

            @pl.loop(0, SLOTS // L)
            def _(g):
                srv = sr_v[pl.ds(g * L, L)]
                for half in range(L // D):
                    t = g * (L // D) + half
                    r0 = srv[half * D]
                    acc = [rows_v[r0, pl.ds(v * L, L)] for v in range(NV)]
                    for d in range(1, D):
                        r = srv[half * D + d]
                        for v in range(NV):
                            acc[v] = acc[v] + rows_v[r, pl.ds(v * L, L)]
                    for v in range(NV):
                        out_v[pl.ds(t * D_MODEL + v * L, L)] = acc[v]

            pltpu.sync_copy(out_v, out_hbm.at[pl.ds(tok0 * D_MODEL,
                                                    C * D_MODEL)])

    return k(ids, tys, lens, key_table, index_table)


@jax.jit
def kernel(path_types, path_ids, path_lengths, key_table, index_table):
    ids = path_ids.reshape(-1).astype(jnp.int32)
    tys = path_types.reshape(-1).astype(jnp.int32)
    lens = jnp.broadcast_to(
        path_lengths.astype(jnp.int32)[..., None], (B, S, D)
    ).reshape(-1)
    out = _sc_pooled(ids, tys, lens,
                     key_table.astype(jnp.float32),
                     index_table.astype(jnp.float32))
    return out.reshape(B, S, D_MODEL)


# speedup vs baseline: 1.0274x; 1.0007x over previous
import functools

import jax
import jax.numpy as jnp
from jax import lax
from jax.experimental import pallas as pl
from jax.experimental.pallas import tpu as pltpu
from jax.experimental.pallas import tpu_sc as plsc

B, S, D = 8, 2048, 8
D_MODEL = 256
BS = B * S

NC, NS, L = 2, 16, 16
NW = NC * NS
TW = BS // NW
C = 16
SLOTS = C * D
NCH = TW // C
NV = D_MODEL // L
ZROW = SLOTS + D_MODEL


def _sc_pooled(ids, tys, lens, key_table, index_table):
    mesh = plsc.VectorSubcoreMesh(core_axis_name="c", subcore_axis_name="s")

    @functools.partial(
        pl.kernel,
        out_type=jax.ShapeDtypeStruct((BS * D_MODEL,), jnp.float32),
        mesh=mesh,
        scratch_types=[
            pltpu.VMEM((SLOTS,), jnp.int32),
            pltpu.VMEM((SLOTS,), jnp.int32),
            pltpu.VMEM((SLOTS,), jnp.int32),
            pltpu.VMEM((SLOTS,), jnp.int32),
            pltpu.VMEM((SLOTS,), jnp.int32),
            pltpu.VMEM((ZROW + 8, D_MODEL), jnp.float32),
            pltpu.VMEM((C * D_MODEL,), jnp.float32),
            pltpu.SemaphoreType.DMA,
        ],
    )
    def k(ids_hbm, tys_hbm, lens_hbm, ktab_hbm, itab_hbm, out_hbm,
          ids_v, tys_v, lens_v, kidx_v, sr_v, rows_v, out_v, gsem):
        wid = lax.axis_index("s") * NC + lax.axis_index("c")
        lane = lax.iota(jnp.int32, L)
        pos = lax.rem(lane, D)
        zeros = jnp.zeros((L,), jnp.float32)
        pltpu.sync_copy(itab_hbm, rows_v.at[pl.ds(SLOTS, D_MODEL)])
        for v in range(NV):
            rows_v[ZROW, pl.ds(v * L, L)] = zeros

        @pl.loop(0, NCH)
        def _(ch):
            tok0 = wid * TW + ch * C
            s0 = tok0 * D
            pltpu.sync_copy(ids_hbm.at[pl.ds(s0, SLOTS)], ids_v)
            pltpu.sync_copy(tys_hbm.at[pl.ds(s0, SLOTS)], tys_v)
            pltpu.sync_copy(lens_hbm.at[pl.ds(s0, SLOTS)], lens_v)

            for g in range(SLOTS // L):
                sl = pl.ds(g * L, L)
                idv = ids_v[sl]
                tyv = tys_v[sl]
                valid = pos < lens_v[sl]
                km = valid & (tyv == 1)
                im = valid & (tyv == 2)
                slot = lane + (g * L)
                kidx_v[sl] = jnp.where(km, idv, 0)
                sr_v[sl] = jnp.where(
                    km, slot,
                    jnp.where(im, SLOTS + jnp.minimum(idv, D_MODEL - 1), ZROW))

            NSTR = 8
            W = SLOTS // NSTR
            cps = [pltpu.async_copy(
                ktab_hbm.at[kidx_v.at[pl.ds(j * W, W)]],
                rows_v.at[pl.ds(j * W, W)], gsem) for j in range(NSTR)]
            for cp in cps:
                cp.wait()

            @pl.loop(0, SLOTS // L)
            def _(g):
                srv = sr_v[pl.ds(g * L, L)]
                for half in range(L // D):
                    t = g * (L // D) + half
                    r0 = srv[half * D]
                    acc = [rows_v[r0, pl.ds(v * L, L)] for v in range(NV)]
                    for d in range(1, D):
                        r = srv[half * D + d]
                        for v in range(NV):
                            acc[v] = acc[v] + rows_v[r, pl.ds(v * L, L)]
                    for v in range(NV):
                        out_v[pl.ds(t * D_MODEL + v * L, L)] = acc[v]

            pltpu.sync_copy(out_v, out_hbm.at[pl.ds(tok0 * D_MODEL,
                                                    C * D_MODEL)])

    return k(ids, tys, lens, key_table, index_table)


@jax.jit
def kernel(path_types, path_ids, path_lengths, key_table, index_table):
    ids = path_ids.reshape(-1).astype(jnp.int32)
    tys = path_types.reshape(-1).astype(jnp.int32)
    lens = jnp.broadcast_to(
        path_lengths.astype(jnp.int32)[..., None], (B, S, D)
    ).reshape(-1)
    out = _sc_pooled(ids, tys, lens,
                     key_table.astype(jnp.float32),
                     index_table.astype(jnp.float32))
    return out.reshape(B, S, D_MODEL)
